# baseline (speedup 1.0000x reference)
import jax
import jax.numpy as jnp
from jax import lax
from jax.experimental import pallas as pl
from jax.experimental.pallas import tpu as pltpu
from jax.experimental.pallas import tpu_sc as plsc

NUM_WORKERS = 32
LANES = 16
BATCH = 16384
EMBED_DIM = 64
NUM_RELATIONS = 1000
B_PER_W = BATCH // NUM_WORKERS
PASS = 128
N_PASS = B_PER_W // PASS


def _transe_body(head_hbm, rel_hbm, epair_hbm, relT_hbm, outT_hbm,
                 hidx, ridx, pidx, relv, ebuf0, ebuf1, ocols0, ocols1,
                 esem, osem):
    wid = lax.axis_index("s") * 2 + lax.axis_index("c")
    base = wid * B_PER_W

    pltpu.sync_copy(head_hbm.at[pl.ds(base, B_PER_W)], hidx)
    pltpu.sync_copy(rel_hbm.at[pl.ds(base, B_PER_W)], ridx)
    pltpu.sync_copy(relT_hbm, relv)

    def shift(i, carry):
        sl = pl.ds(i * LANES, LANES)
        pidx[sl] = jax.lax.shift_right_logical(hidx[sl], 1)
        return carry

    lax.fori_loop(0, B_PER_W // LANES, shift, 0)

    ebufs = [ebuf0, ebuf1]
    obufs = [ocols0, ocols1]
    lane = lax.iota(jnp.int32, LANES)

    def fire(p):
        return pltpu.async_copy(
            epair_hbm.at[pidx.at[pl.ds(p * PASS, PASS)]],
            ebufs[p % 2], esem)

    def compute(p):
        ebuf = ebufs[p % 2]
        ocols = obufs[p % 2]

        def kbody(k, carry):
            hv = hidx[pl.ds(p * PASS + k * LANES, LANES)]
            rv = ridx[pl.ds(p * PASS + k * LANES, LANES)]
            pos = k * LANES + lane
            hcol0 = jax.lax.shift_left(
                jax.lax.bitwise_and(hv, jnp.int32(1)), 6)
            for d in range(EMBED_DIM):
                ev = plsc.load_gather(ebuf, [pos, hcol0 + d])
                rvv = plsc.load_gather(relv, [jnp.full((LANES,), d, jnp.int32), rv])
                ocols[d, pl.ds(k * LANES, LANES)] = ev + rvv
            return carry

        lax.fori_loop(0, PASS // LANES, kbody, 0)
        return pltpu.async_copy(
            ocols, outT_hbm.at[:, pl.ds(base + p * PASS, PASS)], osem)

    writes = []
    cp = fire(0)
    for p in range(N_PASS):
        nxt = fire(p + 1) if p + 1 < N_PASS else None
        cp.wait()
        writes.append(compute(p))
        cp = nxt
    for w in writes:
        w.wait()


@jax.jit
def _transe(head, relation, entity_embeddings, relation_embeddings):
    epair = entity_embeddings.reshape(500000, 2 * EMBED_DIM)
    mesh = plsc.VectorSubcoreMesh(core_axis_name="c", subcore_axis_name="s")
    outT = pl.kernel(
        _transe_body,
        out_type=jax.ShapeDtypeStruct((EMBED_DIM, BATCH), jnp.float32),
        mesh=mesh,
        scratch_types=[
            pltpu.VMEM((B_PER_W,), jnp.int32),
            pltpu.VMEM((B_PER_W,), jnp.int32),
            pltpu.VMEM((B_PER_W,), jnp.int32),
            pltpu.VMEM((EMBED_DIM, NUM_RELATIONS), jnp.float32),
            pltpu.VMEM((PASS, 2 * EMBED_DIM), jnp.float32),
            pltpu.VMEM((PASS, 2 * EMBED_DIM), jnp.float32),
            pltpu.VMEM((EMBED_DIM, PASS), jnp.float32),
            pltpu.VMEM((EMBED_DIM, PASS), jnp.float32),
            pltpu.SemaphoreType.DMA,
            pltpu.SemaphoreType.DMA,
        ],
        compiler_params=pltpu.CompilerParams(needs_layout_passes=False),
    )(head, relation, epair, relation_embeddings.T)
    return outT.T


def kernel(head, relation, entity_embeddings, relation_embeddings):
    return _transe(head, relation, entity_embeddings, relation_embeddings)


# speedup vs baseline: 2.2063x; 2.2063x over previous
import jax
import jax.numpy as jnp
from jax import lax
from jax.experimental import pallas as pl
from jax.experimental.pallas import tpu as pltpu
from jax.experimental.pallas import tpu_sc as plsc

NUM_WORKERS = 32
LANES = 16
BATCH = 16384
EMBED_DIM = 64
NUM_ENT = 1000000
NUM_REL = 1000
CHW = 256
N_FULL = NUM_ENT // CHW
N_CHUNKS = N_FULL + 1
TAIL_START = N_FULL * CHW
TAIL_W = NUM_ENT - TAIL_START
BASE_CH = N_CHUNKS // NUM_WORKERS
EXTRA = N_CHUNKS - BASE_CH * NUM_WORKERS
WIN = 2048
CAP = 1024


def _sread(ref, i):
    return ref[pl.ds(i, LANES)][0]


def _transe_body(head_hbm, rel_hbm, entT_hbm, relT_hbm, tailT_hbm, out_hbm,
                 hwin, rwin, clist, rlist, spk, srl, cc, offs, offs2,
                 cbuf, relv, rows_v, posb):
    wid = lax.axis_index("s") * 2 + lax.axis_index("c")
    n_ch = BASE_CH + jnp.where(wid < EXTRA, 1, 0)
    cstart = wid * BASE_CH + jnp.minimum(wid, EXTRA)
    lo = cstart * CHW
    hi = (cstart + n_ch) * CHW
    lane = lax.iota(jnp.int32, LANES)
    lane0 = lane == 0

    def _swrite(ref, i, val):
        plsc.store_scatter(ref, [jnp.full((LANES,), i, jnp.int32)],
                           jnp.full((LANES,), val, jnp.int32), mask=lane0)

    pltpu.sync_copy(relT_hbm, relv)
    pltpu.sync_copy(tailT_hbm, cbuf.at[:, pl.ds(CHW, 128)])

    count = jnp.int32(0)
    for win in range(BATCH // WIN):
        pltpu.sync_copy(head_hbm.at[pl.ds(win * WIN, WIN)], hwin)
        pltpu.sync_copy(rel_hbm.at[pl.ds(win * WIN, WIN)], rwin)

        def scan_group(g, cnt, _win=win):
            sl = pl.ds(g * LANES, LANES)
            hv = hwin[sl]
            rv = rwin[sl]
            pos = jnp.int32(_win * WIN) + g * LANES + lane
            m = jnp.logical_and(hv >= lo, hv < hi)
            pk = lax.shift_left(hv - lo, 14) + pos
            plsc.store_compressed(clist.at[pl.ds(cnt, LANES)], pk, mask=m)
            plsc.store_compressed(rlist.at[pl.ds(cnt, LANES)], rv, mask=m)
            return cnt + jnp.sum(m.astype(jnp.int32))

        count = lax.fori_loop(0, WIN // LANES, scan_group, count)

    zeros = jnp.zeros((LANES,), jnp.int32)
    for g in range(8):
        cc[pl.ds(g * LANES, LANES)] = zeros

    def count_group(j, carry):
        sl = pl.ds(j * LANES, LANES)
        pkv = clist[sl]
        valid = (j * LANES + lane) < count
        plsc.addupdate_scatter(
            cc, [lax.shift_right_logical(pkv, 22)],
            jnp.ones((LANES,), jnp.int32), mask=valid)
        return carry

    lax.fori_loop(0, CAP // LANES, count_group, 0)

    tot = jnp.int32(0)
    for g in range(8):
        sl = pl.ds(g * LANES, LANES)
        v = cc[sl]
        ex = plsc.cumsum(v) - v + tot
        offs[sl] = ex
        offs2[sl] = ex
        tot = tot + jnp.sum(v)

    def redist(j, carry):
        pk = _sread(clist, j)
        rl = _sread(rlist, j)
        ch = lax.shift_right_logical(pk, 22)
        slot = _sread(offs2, ch)
        _swrite(offs2, ch, slot + 1)
        _swrite(spk, slot, pk)
        _swrite(srl, slot, rl)
        return carry

    lax.fori_loop(0, count, redist, 0)

    def chunk_body(i, carry):
        c = cstart + i
        start = jnp.where(c == N_FULL, jnp.int32(TAIL_START - CHW), c * CHW)

        @pl.when(c != N_FULL)
        def _():
            pltpu.sync_copy(entT_hbm.at[:, pl.ds(start, CHW)],
                            cbuf.at[:, pl.ds(0, CHW)])

        s0 = _sread(offs, i)
        s1 = _sread(offs, i + 1)

        def entry(s, mcarry):
            m, _ = mcarry
            pk = _sread(spk, s)
            rl = _sread(srl, s)
            col = lax.shift_right_logical(pk, 14) + lo - start
            p = lax.bitwise_and(pk, jnp.int32(16383))
            slot = lax.bitwise_and(m, jnp.int32(15))
            colv = jnp.full((LANES,), col, jnp.int32)
            rlv = jnp.full((LANES,), rl, jnp.int32)
            for g in range(EMBED_DIM // LANES):
                dvec = jnp.arange(g * LANES, (g + 1) * LANES, dtype=jnp.int32)
                ev = plsc.load_gather(cbuf, [dvec, colv])
                rvv = plsc.load_gather(relv, [dvec, rlv])
                rows_v[slot, pl.ds(g * LANES, LANES)] = ev + rvv
            plsc.store_scatter(posb, [jnp.zeros((LANES,), jnp.int32),
                                      jnp.full((LANES,), slot, jnp.int32)],
                               jnp.full((LANES,), p, jnp.int32), mask=lane0)

            @pl.when(slot == 15)
            def _():
                pltpu.sync_copy(rows_v, out_hbm.at[posb.at[0]])

            return m + 1, p

        return lax.fori_loop(s0, s1, entry, carry)

    m, lastp = lax.fori_loop(0, n_ch, chunk_body,
                             (jnp.int32(0), jnp.int32(0)))

    rem = lax.bitwise_and(m, jnp.int32(15))

    @pl.when(rem != 0)
    def _():
        def pad(g, carry):
            plsc.store_scatter(posb, [jnp.zeros((LANES,), jnp.int32),
                                      jnp.full((LANES,), g, jnp.int32)],
                               jnp.full((LANES,), lastp, jnp.int32),
                               mask=lane0)
            for q in range(EMBED_DIM // LANES):
                sl = pl.ds(q * LANES, LANES)
                src = plsc.load_gather(
                    rows_v, [jnp.full((LANES,), rem - 1, jnp.int32),
                             q * LANES + lane])
                plsc.store_scatter(
                    rows_v, [jnp.full((LANES,), g, jnp.int32),
                             q * LANES + lane], src)
            return carry

        lax.fori_loop(rem, 16, pad, 0)
        pltpu.sync_copy(rows_v, out_hbm.at[posb.at[0]])


@jax.jit
def _transe(head, relation, entity_embeddings, relation_embeddings):
    mesh = plsc.VectorSubcoreMesh(core_axis_name="c", subcore_axis_name="s")
    out_wide = pl.kernel(
        _transe_body,
        out_type=jax.ShapeDtypeStruct((BATCH, 2 * EMBED_DIM), jnp.float32),
        mesh=mesh,
        scratch_types=[
            pltpu.VMEM((WIN,), jnp.int32),
            pltpu.VMEM((WIN,), jnp.int32),
            pltpu.VMEM((CAP + LANES,), jnp.int32),
            pltpu.VMEM((CAP + LANES,), jnp.int32),
            pltpu.VMEM((CAP + LANES,), jnp.int32),
            pltpu.VMEM((CAP + LANES,), jnp.int32),
            pltpu.VMEM((128,), jnp.int32),
            pltpu.VMEM((128 + LANES,), jnp.int32),
            pltpu.VMEM((128 + LANES,), jnp.int32),
            pltpu.VMEM((EMBED_DIM, CHW + 128), jnp.float32),
            pltpu.VMEM((EMBED_DIM, 1024), jnp.float32),
            pltpu.VMEM((LANES, 2 * EMBED_DIM), jnp.float32),
            pltpu.VMEM((1, LANES), jnp.int32),
        ],
        compiler_params=pltpu.CompilerParams(needs_layout_passes=False),
    )(head, relation, entity_embeddings.T,
      jnp.pad(relation_embeddings.T, ((0, 0), (0, 1024 - NUM_REL))),
      jnp.pad(entity_embeddings.T[:, TAIL_START:], ((0, 0), (0, TAIL_W))))
    return out_wide[:, :EMBED_DIM]


def kernel(head, relation, entity_embeddings, relation_embeddings):
    return _transe(head, relation, entity_embeddings, relation_embeddings)


# speedup vs baseline: 3.3538x; 1.5201x over previous
import jax
import jax.numpy as jnp
from jax import lax
from jax.experimental import pallas as pl
from jax.experimental.pallas import tpu as pltpu
from jax.experimental.pallas import tpu_sc as plsc

NUM_WORKERS = 32
LANES = 16
BATCH = 16384
EMBED_DIM = 64
NUM_ENT = 1000000
NUM_REL = 1000
CHW = 256
N_FULL = NUM_ENT // CHW
N_CHUNKS = N_FULL + 1
TAIL_START = N_FULL * CHW
TAIL_W = NUM_ENT - TAIL_START
BASE_CH = N_CHUNKS // NUM_WORKERS
EXTRA = N_CHUNKS - BASE_CH * NUM_WORKERS
WIN = 2048
CAP = 1024
PARK = 2 * CHW


def _sread(ref, i):
    return ref[pl.ds(i, LANES)][0]


def _transe_body(head_hbm, rel_hbm, entT_hbm, relT_hbm, tailT_hbm, out_hbm,
                 hwin, rwin, clist, rlist, spk, srl, cc, offs, offs2,
                 cbuf, relv, rows_v, posb, csem, osem):
    wid = lax.axis_index("s") * 2 + lax.axis_index("c")
    n_ch = BASE_CH + jnp.where(wid < EXTRA, 1, 0)
    cstart = wid * BASE_CH + jnp.minimum(wid, EXTRA)
    lo = cstart * CHW
    hi = (cstart + n_ch) * CHW
    lane = lax.iota(jnp.int32, LANES)
    lane0 = lane == 0

    def _swrite(ref, i, val):
        plsc.store_scatter(ref, [jnp.full((LANES,), i, jnp.int32)],
                           jnp.full((LANES,), val, jnp.int32), mask=lane0)

    pltpu.sync_copy(relT_hbm, relv)
    pltpu.sync_copy(tailT_hbm, cbuf.at[:, pl.ds(PARK, 128)])

    count = jnp.int32(0)
    for win in range(BATCH // WIN):
        pltpu.sync_copy(head_hbm.at[pl.ds(win * WIN, WIN)], hwin)
        pltpu.sync_copy(rel_hbm.at[pl.ds(win * WIN, WIN)], rwin)

        def scan_group(g, cnt, _win=win):
            sl = pl.ds(g * LANES, LANES)
            hv = hwin[sl]
            rv = rwin[sl]
            pos = jnp.int32(_win * WIN) + g * LANES + lane
            m = jnp.logical_and(hv >= lo, hv < hi)
            pk = lax.shift_left(hv - lo, 14) + pos
            plsc.store_compressed(clist.at[pl.ds(cnt, LANES)], pk, mask=m)
            plsc.store_compressed(rlist.at[pl.ds(cnt, LANES)], rv, mask=m)
            return cnt + jnp.sum(m.astype(jnp.int32))

        count = lax.fori_loop(0, WIN // LANES, scan_group, count)

    zeros = jnp.zeros((LANES,), jnp.int32)
    for g in range(8):
        cc[pl.ds(g * LANES, LANES)] = zeros

    def count_group(j, carry):
        sl = pl.ds(j * LANES, LANES)
        pkv = clist[sl]
        valid = (j * LANES + lane) < count
        plsc.addupdate_scatter(
            cc, [lax.shift_right_logical(pkv, 22)],
            jnp.ones((LANES,), jnp.int32), mask=valid)
        return carry

    lax.fori_loop(0, CAP // LANES, count_group, 0)

    tot = jnp.int32(0)
    for g in range(8):
        sl = pl.ds(g * LANES, LANES)
        v = cc[sl]
        ex = plsc.cumsum(v) - v + tot
        offs[sl] = ex
        offs2[sl] = ex
        tot = tot + jnp.sum(v)

    def redist(j, carry):
        pk = _sread(clist, j)
        rl = _sread(rlist, j)
        ch = lax.shift_right_logical(pk, 22)
        slot = _sread(offs2, ch)
        _swrite(offs2, ch, slot + 1)
        _swrite(spk, slot, pk)
        _swrite(srl, slot, rl)
        return carry

    lax.fori_loop(0, count, redist, 0)

    n_full_local = n_ch - jnp.where(wid == NUM_WORKERS - 1, 1, 0)

    def nonempty(i):
        return _sread(offs, i + 1) > _sread(offs, i)

    def fire(i):
        @pl.when(jnp.logical_and(i < n_full_local, nonempty(i)))
        def _():
            off = lax.bitwise_and(i, jnp.int32(1)) * CHW
            pltpu.async_copy(
                entT_hbm.at[:, pl.ds((cstart + i) * CHW, CHW)],
                cbuf.at[:, pl.ds(off, CHW)], csem)

    def drain(i):
        @pl.when(nonempty(i))
        def _():
            pltpu.make_async_copy(
                entT_hbm.at[:, pl.ds(0, CHW)],
                cbuf.at[:, pl.ds(0, CHW)], csem).wait()

    def extract(s0, s1, colbase, mf):
        def entry(s, mf):
            m, f, _ = mf
            pk = _sread(spk, s)
            rl = _sread(srl, s)
            col = colbase + lax.shift_right_logical(pk, 14)
            p = lax.bitwise_and(pk, jnp.int32(16383))
            slot = lax.bitwise_and(m, jnp.int32(15))
            bank = lax.bitwise_and(f, jnp.int32(1))
            row = bank * LANES + slot
            colv = jnp.full((LANES,), col, jnp.int32)
            rlv = jnp.full((LANES,), rl, jnp.int32)
            rowv = jnp.full((LANES,), row, jnp.int32)
            for g in range(EMBED_DIM // LANES):
                dvec = jnp.arange(g * LANES, (g + 1) * LANES, dtype=jnp.int32)
                ev = plsc.load_gather(cbuf, [dvec, colv])
                rvv = plsc.load_gather(relv, [dvec, rlv])
                plsc.store_scatter(rows_v, [rowv, g * LANES + lane], ev + rvv)
            plsc.store_scatter(posb, [jnp.full((LANES,), bank, jnp.int32),
                                      jnp.full((LANES,), slot, jnp.int32)],
                               jnp.full((LANES,), p, jnp.int32), mask=lane0)

            @pl.when(slot == 15)
            def _():
                @pl.when(f > 0)
                def _():
                    pltpu.make_async_copy(
                        rows_v.at[pl.ds(0, LANES)],
                        out_hbm.at[pl.ds(0, LANES)], osem).wait()

                pltpu.async_copy(rows_v.at[pl.ds(bank * LANES, LANES)],
                                 out_hbm.at[posb.at[bank]], osem)

            return (m + 1,
                    f + jnp.where(slot == 15, 1, 0).astype(jnp.int32), p)

        return lax.fori_loop(s0, s1, entry, mf)

    def chunk_body(i, mf):
        fire(i + 1)
        drain(i)
        off = lax.bitwise_and(i, jnp.int32(1)) * CHW
        s0 = _sread(offs, i)
        s1 = _sread(offs, i + 1)
        return extract(s0, s1, off - i * CHW, mf)

    def _finish(mf):
        m, f, lastp = mf
        rem = lax.bitwise_and(m, jnp.int32(15))
        bank = lax.bitwise_and(f, jnp.int32(1))

        @pl.when(f > 0)
        def _():
            pltpu.make_async_copy(rows_v.at[pl.ds(0, LANES)],
                                  out_hbm.at[pl.ds(0, LANES)], osem).wait()

        @pl.when(rem != 0)
        def _():
            def pad(g, carry):
                plsc.store_scatter(
                    posb, [jnp.full((LANES,), bank, jnp.int32),
                           jnp.full((LANES,), g, jnp.int32)],
                    jnp.full((LANES,), lastp, jnp.int32), mask=lane0)
                rowg = bank * LANES + g
                for q in range(EMBED_DIM // LANES):
                    src = plsc.load_gather(
                        rows_v, [jnp.full((LANES,), bank * LANES + rem - 1,
                                          jnp.int32), q * LANES + lane])
                    plsc.store_scatter(
                        rows_v, [jnp.full((LANES,), rowg, jnp.int32),
                                 q * LANES + lane], src)
                return carry

            lax.fori_loop(rem, 16, pad, 0)
            pltpu.sync_copy(rows_v.at[pl.ds(bank * LANES, LANES)],
                            out_hbm.at[posb.at[bank]])

    fire(jnp.int32(0))
    mf = lax.fori_loop(0, n_full_local, chunk_body,
                       (jnp.int32(0), jnp.int32(0), jnp.int32(0)))

    @pl.when(wid == NUM_WORKERS - 1)
    def _():
        s0 = _sread(offs, n_full_local)
        _finish(extract(s0, count, PARK - n_full_local * CHW, mf))

    @pl.when(wid != NUM_WORKERS - 1)
    def _():
        _finish(mf)


@jax.jit
def _transe(head, relation, entity_embeddings, relation_embeddings):
    mesh = plsc.VectorSubcoreMesh(core_axis_name="c", subcore_axis_name="s")
    out_wide = pl.kernel(
        _transe_body,
        out_type=jax.ShapeDtypeStruct((BATCH, 2 * EMBED_DIM), jnp.float32),
        mesh=mesh,
        scratch_types=[
            pltpu.VMEM((WIN,), jnp.int32),
            pltpu.VMEM((WIN,), jnp.int32),
            pltpu.VMEM((CAP + LANES,), jnp.int32),
            pltpu.VMEM((CAP + LANES,), jnp.int32),
            pltpu.VMEM((CAP + LANES,), jnp.int32),
            pltpu.VMEM((CAP + LANES,), jnp.int32),
            pltpu.VMEM((128,), jnp.int32),
            pltpu.VMEM((128 + LANES,), jnp.int32),
            pltpu.VMEM((128 + LANES,), jnp.int32),
            pltpu.VMEM((EMBED_DIM, PARK + 128), jnp.float32),
            pltpu.VMEM((EMBED_DIM, 1024), jnp.float32),
            pltpu.VMEM((2 * LANES, 2 * EMBED_DIM), jnp.float32),
            pltpu.VMEM((2, LANES), jnp.int32),
            pltpu.SemaphoreType.DMA,
            pltpu.SemaphoreType.DMA,
        ],
        compiler_params=pltpu.CompilerParams(needs_layout_passes=False),
    )(head, relation, entity_embeddings.T,
      jnp.pad(relation_embeddings.T, ((0, 0), (0, 1024 - NUM_REL))),
      jnp.pad(entity_embeddings.T[:, TAIL_START:], ((0, 0), (0, TAIL_W))))
    return out_wide[:, :EMBED_DIM]


def kernel(head, relation, entity_embeddings, relation_embeddings):
    return _transe(head, relation, entity_embeddings, relation_embeddings)
